# baseline (speedup 1.0000x reference)
import functools

import jax
import jax.numpy as jnp
from jax import lax
from jax.experimental import pallas as pl
from jax.experimental.pallas import tpu as pltpu
from jax.experimental.pallas import tpu_sc as plsc

N = 100000
NC, NS, L = 2, 16, 16
W = NC * NS
PW = 3136
NP = W * PW
CELLS = 1 << 22
NCH = 64
CH = CELLS // NCH
BLK = NP // 16
FLUSH = 8192
BLKC = 2048
SEGV = 256
SEGC = SEGV * L
PAD_HC = 0x7FFFFFF0

_MESH = plsc.VectorSubcoreMesh(
    core_axis_name="c", subcore_axis_name="s", num_cores=NC, num_subcores=NS)
_PARAMS = pltpu.CompilerParams(needs_layout_passes=False)


def _al8(x):
    return pl.multiple_of(x, 8)


def _wid():
    return lax.axis_index("s") * NC + lax.axis_index("c")


def _lane():
    return lax.iota(jnp.int32, L)


@functools.partial(
    pl.kernel,
    out_type=(
        jax.ShapeDtypeStruct((CELLS,), jnp.int32),
        jax.ShapeDtypeStruct((CELLS,), jnp.float32),
        jax.ShapeDtypeStruct((NCH * L,), jnp.int32),
        jax.ShapeDtypeStruct((2 * NP,), jnp.int32),
    ),
    mesh=_MESH,
    compiler_params=_PARAMS,
    scratch_types=[
        pltpu.VMEM((CH,), jnp.float32),
        pltpu.VMEM((BLK,), jnp.int32),
        pltpu.VMEM((BLK,), jnp.float32),
        pltpu.VMEM((BLK,), jnp.int32),
        pltpu.VMEM((BLK,), jnp.float32),
        pltpu.VMEM((BLK + L,), jnp.int32),
        pltpu.VMEM((BLK + L,), jnp.float32),
        pltpu.VMEM((FLUSH + SEGC + L,), jnp.int32),
        pltpu.VMEM((FLUSH + SEGC + L,), jnp.float32),
        pltpu.VMEM((L,), jnp.int32),
        pltpu.SemaphoreType.DMA,
        pltpu.SemaphoreType.DMA,
    ],
)
def _pool_kernel(coords_hbm, f_hbm, neg_hbm, cells_hbm, vals_hbm, counts_hbm,
                 hc2_hbm, table, hbuf0, fbuf0, hbuf1, fbuf1, fxc, fxv,
                 ocell, oval, cntbuf, sem0, sem1):
    w = _wid()
    lane = _lane()
    sid = lax.axis_index("s")
    cid = lax.axis_index("c")
    hcbase = cid * NP

    HP = NP // NS
    HS = HP // 4
    for s in range(4):
        pbase = sid * HP + s * HS
        pltpu.sync_copy(coords_hbm.at[pl.ds(_al8(pbase * 4), HS * 4)],
                        hbuf0.at[pl.ds(0, HS * 4)])

        def hbody(j, hcarry):
            fi = (j * L + lane) * 4
            b = plsc.load_gather(hbuf0, [fi])
            x = plsc.load_gather(hbuf0, [fi + 1])
            y = plsc.load_gather(hbuf0, [fi + 2])
            z = plsc.load_gather(hbuf0, [fi + 3])
            hc = (b << 15) | ((x >> 2) << 10) | ((y >> 2) << 5) | (z >> 2)
            row = pbase + j * L + lane
            hc = jnp.where(row < N, hc, jnp.int32(PAD_HC))
            hbuf1[pl.ds(j * L, L)] = hc
            return hcarry

        lax.fori_loop(0, HS // L, hbody, jnp.int32(0))
        pltpu.sync_copy(hbuf1.at[pl.ds(0, HS)],
                        hc2_hbm.at[pl.ds(_al8(hcbase + pbase), HS)])
    plsc.subcore_barrier()

    for t in range(2):
        c = w * 2 + t
        cbase = c * CH
        pltpu.sync_copy(neg_hbm, table)

        def _start(blk, hb, fb, sem):
            pltpu.async_copy(hc2_hbm.at[pl.ds(_al8(hcbase + blk * BLK), BLK)], hb, sem)
            pltpu.async_copy(f_hbm.at[pl.ds(_al8(blk * BLK), BLK)], fb, sem)

        def _wait(blk, hb, fb, sem):
            pltpu.make_async_copy(
                hc2_hbm.at[pl.ds(_al8(hcbase + blk * BLK), BLK)], hb, sem).wait()
            pltpu.make_async_copy(
                f_hbm.at[pl.ds(_al8(blk * BLK), BLK)], fb, sem).wait()

        def _process(hb, fb):
            @plsc.parallel_loop(0, BLK // L, step=1, unroll=4)
            def p12(j):
                h = hb[pl.ds(j * L, L)]
                v = fb[pl.ds(j * L, L)]
                inr = (h >> 16) == c
                li = jnp.where(inr, h & 0xFFFF, 0)
                cur = plsc.load_gather(table, [li], mask=inr)
                need = inr & (v > cur)
                plsc.store_scatter(table, [li], v, mask=need)

            @plsc.parallel_loop(0, BLK // L, step=1, unroll=2,
                                carry=jnp.int32(0))
            def p3(j, fcnt):
                h = hb[pl.ds(j * L, L)]
                v = fb[pl.ds(j * L, L)]
                inr = (h >> 16) == c
                li = jnp.where(inr, h & 0xFFFF, 0)
                cur = plsc.load_gather(table, [li], mask=inr)
                lost = inr & (v > cur)
                plsc.store_compressed(fxc.at[pl.ds(fcnt, L)], li, mask=lost)
                plsc.store_compressed(fxv.at[pl.ds(fcnt, L)], v, mask=lost)
                pc = plsc.all_reduce_population_count(lost)
                return fcnt + pc[0]

            fcnt = p3

            def drain(nfv):
                def dvbody(q, vcarry):
                    m = (q * L + lane) < fcnt
                    li = fxc[pl.ds(q * L, L)]
                    v = fxv[pl.ds(q * L, L)]
                    li = jnp.where(m, li, 0)

                    def wcond(pend):
                        return pend

                    def wbody(pend):
                        cur = plsc.load_gather(table, [li], mask=m)
                        need = m & (v > cur)
                        plsc.store_scatter(table, [li], v, mask=need)
                        cur2 = plsc.load_gather(table, [li], mask=m)
                        return jnp.any(m & (v > cur2))

                    lax.while_loop(wcond, wbody, jnp.any(m))
                    return vcarry

                lax.fori_loop(0, nfv, dvbody, jnp.int32(0))
                return jnp.int32(0)

            lax.cond(fcnt > 0, drain, lambda a: a, (fcnt + L - 1) // L)

        NPAIR = NP // BLK // 2
        _start(0, hbuf0, fbuf0, sem0)

        def pairbody(p, carry):
            b0 = 2 * p
            _wait(b0, hbuf0, fbuf0, sem0)
            _start(b0 + 1, hbuf1, fbuf1, sem1)
            _process(hbuf0, fbuf0)
            _wait(b0 + 1, hbuf1, fbuf1, sem1)

            @pl.when(p < NPAIR - 1)
            def _():
                _start(b0 + 2, hbuf0, fbuf0, sem0)

            _process(hbuf1, fbuf1)
            return carry

        lax.fori_loop(0, NPAIR, pairbody, jnp.int32(0))

        def segloop(s, carry):
            off0, flushed0 = carry

            @plsc.parallel_loop(0, SEGV, step=1, unroll=4, carry=off0)
            def seg(i, off_):
                idx = s * SEGV + i
                tv = table[pl.ds(idx * L, L)]
                occ = tv > -0.5
                cells = cbase + idx * L + lane
                plsc.store_compressed(ocell.at[pl.ds(off_, L)], cells,
                                      mask=occ)
                plsc.store_compressed(oval.at[pl.ds(off_, L)], tv, mask=occ)
                pc = plsc.all_reduce_population_count(occ)
                return off_ + pc[0]

            def do_flush(args):
                off_, flushed_ = args
                pltpu.sync_copy(ocell.at[pl.ds(0, FLUSH)],
                                cells_hbm.at[pl.ds(_al8(cbase + flushed_), FLUSH)])
                pltpu.sync_copy(oval.at[pl.ds(0, FLUSH)],
                                vals_hbm.at[pl.ds(_al8(cbase + flushed_), FLUSH)])
                rem = off_ - FLUSH

                def mv(q, mcarry):
                    tc = ocell[pl.ds(FLUSH + q * L, L)]
                    tvv = oval[pl.ds(FLUSH + q * L, L)]
                    ocell[pl.ds(q * L, L)] = tc
                    oval[pl.ds(q * L, L)] = tvv
                    return mcarry

                lax.fori_loop(0, (rem + L - 1) // L, mv, jnp.int32(0))
                return (rem, flushed_ + FLUSH)

            return lax.cond(seg >= FLUSH, do_flush, lambda a: a,
                            (seg, flushed0))

        off, flushed = lax.fori_loop(
            0, (CH // L) // SEGV, segloop, (jnp.int32(0), jnp.int32(0)))

        def final_flush(args):
            off_, flushed_ = args
            pltpu.sync_copy(ocell.at[pl.ds(0, FLUSH)],
                            cells_hbm.at[pl.ds(_al8(cbase + flushed_), FLUSH)])
            pltpu.sync_copy(oval.at[pl.ds(0, FLUSH)],
                            vals_hbm.at[pl.ds(_al8(cbase + flushed_), FLUSH)])
            return args

        lax.cond(off > 0, final_flush, lambda a: a, (off, flushed))

        total = off + flushed
        cntbuf[pl.ds(0, L)] = jnp.full((L,), 1, jnp.int32) * total
        pltpu.sync_copy(cntbuf, counts_hbm.at[pl.ds(_al8(c * L), L)])


@functools.partial(
    pl.kernel,
    out_type=(
        jax.ShapeDtypeStruct((NP,), jnp.float32),
        jax.ShapeDtypeStruct((4 * NP,), jnp.int32),
        jax.ShapeDtypeStruct((8,), jnp.int32),
    ),
    mesh=_MESH,
    compiler_params=_PARAMS,
    scratch_types=[
        pltpu.VMEM((NCH * L,), jnp.int32),
        pltpu.VMEM((NCH,), jnp.int32),
        pltpu.VMEM((PW,), jnp.int32),
        pltpu.VMEM((PW,), jnp.float32),
        pltpu.VMEM((PW,), jnp.int32),
        pltpu.VMEM((PW * 4,), jnp.int32),
        pltpu.VMEM((L,), jnp.int32),
    ],
)
def _place_kernel(cells_hbm, vals_hbm, counts_hbm,
                  feats_hbm, coords_hbm, total_hbm,
                  cbuf, offsv, sidx, fblk, cg, c4, totbuf):
    w = _wid()
    lane = _lane()
    salt = w * 64

    pltpu.sync_copy(counts_hbm, cbuf)

    carry = jnp.int32(0)
    for k in range(NCH // L):
        cidx = (k * L + lane) * L
        cnt = plsc.load_gather(cbuf, [cidx])
        inc = plsc.cumsum(cnt)
        offsv[pl.ds(k * L, L)] = inc - cnt + carry
        carry = carry + inc[L - 1]
    tot = carry

    @pl.when(w == 0)
    def _():
        totbuf[pl.ds(0, L)] = jnp.full((L,), 1, jnp.int32) * tot
        pltpu.sync_copy(totbuf.at[pl.ds(0, 8)], total_hbm)

    base = w * PW

    def rbody(j, vcarry):
        p = base + j * L + lane
        lo = jnp.zeros((L,), jnp.int32)
        for step in (32, 16, 8, 4, 2, 1):
            cand = lo + step
            ov = plsc.load_gather(offsv, [jnp.minimum(cand, NCH - 1)])
            ok = (cand <= NCH - 1) & (ov <= p)
            lo = jnp.where(ok, cand, lo)
        obase = plsc.load_gather(offsv, [lo])
        s = lo * CH + (p - obase)
        s = jnp.where(p < tot, s, (p + salt) & 2047)
        sidx[pl.ds(j * L, L)] = s
        return vcarry

    lax.fori_loop(0, PW // L, rbody, jnp.int32(0))

    pltpu.sync_copy(cells_hbm.at[sidx], cg)
    pltpu.sync_copy(vals_hbm.at[sidx], fblk)

    def dbody(j, vcarry):
        p = base + j * L + lane
        live = p < tot
        cell = cg[pl.ds(j * L, L)]
        val = fblk[pl.ds(j * L, L)]
        fblk[pl.ds(j * L, L)] = jnp.where(live, val, 0.0)
        cell = jnp.where(live, cell, 0)
        p4 = 4 * (j * L + lane)
        plsc.store_scatter(c4, [p4], cell >> 15)
        plsc.store_scatter(c4, [p4 + 1], (cell >> 10) & 31)
        plsc.store_scatter(c4, [p4 + 2], (cell >> 5) & 31)
        plsc.store_scatter(c4, [p4 + 3], cell & 31)
        return vcarry

    lax.fori_loop(0, PW // L, dbody, jnp.int32(0))

    pltpu.sync_copy(fblk, feats_hbm.at[pl.ds(_al8(base), PW)])
    pltpu.sync_copy(c4, coords_hbm.at[pl.ds(_al8(base * 4), PW * 4)])


def kernel(ghost_coords, ghost_feats, tensor_stride):
    del tensor_stride
    coords = ghost_coords.astype(jnp.int32)
    feats = ghost_feats.reshape(N).astype(jnp.float32)
    coords_flat = jnp.concatenate(
        [coords.reshape(4 * N), jnp.zeros((4 * (NP - N),), jnp.int32)])
    feats_p = jnp.concatenate([feats, jnp.zeros((NP - N,), jnp.float32)])

    neg = jnp.full((CH,), -1.0, jnp.float32)
    cells, vals, counts, _ = _pool_kernel(coords_flat, feats_p, neg)
    feats_pad, coords_pad, total = _place_kernel(cells, vals, counts)

    tot = total[0]
    feats_o = feats_pad[:N].reshape(N, 1)
    coords_o = coords_pad[:4 * N].reshape(N, 4)
    valid = jnp.arange(N, dtype=jnp.int32) < tot
    return feats_o, coords_o, valid


# speedup vs baseline: 1.4086x; 1.4086x over previous
import functools

import jax
import jax.numpy as jnp
from jax import lax
from jax.experimental import pallas as pl
from jax.experimental.pallas import tpu as pltpu
from jax.experimental.pallas import tpu_sc as plsc

N = 100000
NC, NS, L = 2, 16, 16
W = NC * NS
PW = 3136
CELLS = 1 << 22
NCH = 64
CH = CELLS // NCH
BLK = 4000
NBLK = N // BLK
FLUSH = 8192
SEGV = 256
SEGC = SEGV * L
HS = 1568
HP = 4 * HS

_MESH = plsc.VectorSubcoreMesh(
    core_axis_name="c", subcore_axis_name="s", num_cores=NC, num_subcores=NS)
_PARAMS = pltpu.CompilerParams(needs_layout_passes=False)


def _al8(x):
    return pl.multiple_of(x, 8)


def _wid():
    return lax.axis_index("s") * NC + lax.axis_index("c")


def _lane():
    return lax.iota(jnp.int32, L)


@functools.partial(
    pl.kernel,
    out_type=(
        jax.ShapeDtypeStruct((CELLS,), jnp.int32),
        jax.ShapeDtypeStruct((CELLS,), jnp.float32),
        jax.ShapeDtypeStruct((NCH * L,), jnp.int32),
        jax.ShapeDtypeStruct((2 * N,), jnp.int32),
    ),
    mesh=_MESH,
    compiler_params=_PARAMS,
    scratch_types=[
        pltpu.VMEM((CH,), jnp.float32),
        pltpu.VMEM((HS * 4,), jnp.int32),
        pltpu.VMEM((HS,), jnp.int32),
        pltpu.VMEM((BLK,), jnp.int32),
        pltpu.VMEM((BLK,), jnp.float32),
        pltpu.VMEM((BLK,), jnp.int32),
        pltpu.VMEM((BLK,), jnp.float32),
        pltpu.VMEM((BLK + L,), jnp.int32),
        pltpu.VMEM((BLK + L,), jnp.float32),
        pltpu.VMEM((FLUSH + SEGC + L,), jnp.int32),
        pltpu.VMEM((FLUSH + SEGC + L,), jnp.float32),
        pltpu.VMEM((L,), jnp.int32),
        pltpu.SemaphoreType.DMA,
        pltpu.SemaphoreType.DMA,
    ],
)
def _pool_kernel(coords_hbm, f_hbm, neg_hbm, cells_hbm, vals_hbm, counts_hbm,
                 hc2_hbm, table, cslab, hslab,
                 hbuf0, fbuf0, hbuf1, fbuf1,
                 fxc, fxv, ocell, oval, cntbuf, sem0, sem1):
    w = _wid()
    lane = _lane()
    sid = lax.axis_index("s")
    cid = lax.axis_index("c")
    hcbase = cid * N

    for s in range(4):
        rbase = jnp.minimum(sid * HP + s * HS, N - HS)
        pltpu.sync_copy(coords_hbm.at[pl.ds(_al8(rbase * 4), HS * 4)], cslab)

        def hbody(j, hcarry):
            fi = (j * L + lane) * 4
            b = plsc.load_gather(cslab, [fi])
            x = plsc.load_gather(cslab, [fi + 1])
            y = plsc.load_gather(cslab, [fi + 2])
            z = plsc.load_gather(cslab, [fi + 3])
            hc = (b << 15) | ((x >> 2) << 10) | ((y >> 2) << 5) | (z >> 2)
            hslab[pl.ds(j * L, L)] = hc
            return hcarry

        lax.fori_loop(0, HS // L, hbody, jnp.int32(0))
        pltpu.sync_copy(hslab, hc2_hbm.at[pl.ds(_al8(hcbase + rbase), HS)])
    plsc.subcore_barrier()

    for t in range(2):
        c = w * 2 + t
        cbase = c * CH
        pltpu.sync_copy(neg_hbm, table)

        def _start(blk, hb, fb, sem):
            pltpu.async_copy(
                hc2_hbm.at[pl.ds(_al8(hcbase + blk * BLK), BLK)], hb, sem)
            pltpu.async_copy(f_hbm.at[pl.ds(_al8(blk * BLK), BLK)], fb, sem)

        def _wait(blk, hb, fb, sem):
            pltpu.make_async_copy(
                hc2_hbm.at[pl.ds(_al8(hcbase + blk * BLK), BLK)], hb,
                sem).wait()
            pltpu.make_async_copy(
                f_hbm.at[pl.ds(_al8(blk * BLK), BLK)], fb, sem).wait()

        def _process(hb, fb):
            @plsc.parallel_loop(0, BLK // L, step=1, unroll=4)
            def p12(j):
                h = hb[pl.ds(j * L, L)]
                v = fb[pl.ds(j * L, L)]
                inr = (h >> 16) == c
                li = jnp.where(inr, h & 0xFFFF, 0)
                cur = plsc.load_gather(table, [li], mask=inr)
                need = inr & (v > cur)
                plsc.store_scatter(table, [li], v, mask=need)

            @plsc.parallel_loop(0, BLK // L, step=1, unroll=2,
                                carry=jnp.int32(0))
            def p3(j, fcnt):
                h = hb[pl.ds(j * L, L)]
                v = fb[pl.ds(j * L, L)]
                inr = (h >> 16) == c
                li = jnp.where(inr, h & 0xFFFF, 0)
                cur = plsc.load_gather(table, [li], mask=inr)
                lost = inr & (v > cur)
                plsc.store_compressed(fxc.at[pl.ds(fcnt, L)], li, mask=lost)
                plsc.store_compressed(fxv.at[pl.ds(fcnt, L)], v, mask=lost)
                pc = plsc.all_reduce_population_count(lost)
                return fcnt + pc[0]

            fcnt = p3

            def drain(nfv):
                def dvbody(q, vcarry):
                    m = (q * L + lane) < fcnt
                    li = fxc[pl.ds(q * L, L)]
                    v = fxv[pl.ds(q * L, L)]
                    li = jnp.where(m, li, 0)

                    def wcond(pend):
                        return pend

                    def wbody(pend):
                        cur = plsc.load_gather(table, [li], mask=m)
                        need = m & (v > cur)
                        plsc.store_scatter(table, [li], v, mask=need)
                        cur2 = plsc.load_gather(table, [li], mask=m)
                        return jnp.any(m & (v > cur2))

                    lax.while_loop(wcond, wbody, jnp.any(m))
                    return vcarry

                lax.fori_loop(0, nfv, dvbody, jnp.int32(0))
                return jnp.int32(0)

            lax.cond(fcnt > 0, drain, lambda a: a, (fcnt + L - 1) // L)

        NPAIR = (NBLK - 1) // 2
        _start(0, hbuf0, fbuf0, sem0)

        def pairbody(p, carry):
            b0 = 2 * p
            _wait(b0, hbuf0, fbuf0, sem0)
            _start(b0 + 1, hbuf1, fbuf1, sem1)
            _process(hbuf0, fbuf0)
            _wait(b0 + 1, hbuf1, fbuf1, sem1)
            _start(b0 + 2, hbuf0, fbuf0, sem0)
            _process(hbuf1, fbuf1)
            return carry

        lax.fori_loop(0, NPAIR, pairbody, jnp.int32(0))
        _wait(NBLK - 1, hbuf0, fbuf0, sem0)
        _process(hbuf0, fbuf0)

        def segloop(s, carry):
            off0, flushed0 = carry

            @plsc.parallel_loop(0, SEGV, step=1, unroll=4, carry=off0)
            def seg(i, off_):
                idx = s * SEGV + i
                tv = table[pl.ds(idx * L, L)]
                occ = tv > -0.5
                cells = cbase + idx * L + lane
                plsc.store_compressed(ocell.at[pl.ds(off_, L)], cells,
                                      mask=occ)
                plsc.store_compressed(oval.at[pl.ds(off_, L)], tv, mask=occ)
                pc = plsc.all_reduce_population_count(occ)
                return off_ + pc[0]

            def do_flush(args):
                off_, flushed_ = args
                pltpu.sync_copy(ocell.at[pl.ds(0, FLUSH)],
                                cells_hbm.at[pl.ds(_al8(cbase + flushed_),
                                                   FLUSH)])
                pltpu.sync_copy(oval.at[pl.ds(0, FLUSH)],
                                vals_hbm.at[pl.ds(_al8(cbase + flushed_),
                                                  FLUSH)])
                rem = off_ - FLUSH

                def mv(q, mcarry):
                    tc = ocell[pl.ds(FLUSH + q * L, L)]
                    tvv = oval[pl.ds(FLUSH + q * L, L)]
                    ocell[pl.ds(q * L, L)] = tc
                    oval[pl.ds(q * L, L)] = tvv
                    return mcarry

                lax.fori_loop(0, (rem + L - 1) // L, mv, jnp.int32(0))
                return (rem, flushed_ + FLUSH)

            return lax.cond(seg >= FLUSH, do_flush, lambda a: a,
                            (seg, flushed0))

        off, flushed = lax.fori_loop(
            0, (CH // L) // SEGV, segloop, (jnp.int32(0), jnp.int32(0)))

        def final_flush(args):
            off_, flushed_ = args
            pltpu.sync_copy(ocell.at[pl.ds(0, FLUSH)],
                            cells_hbm.at[pl.ds(_al8(cbase + flushed_), FLUSH)])
            pltpu.sync_copy(oval.at[pl.ds(0, FLUSH)],
                            vals_hbm.at[pl.ds(_al8(cbase + flushed_), FLUSH)])
            return args

        lax.cond(off > 0, final_flush, lambda a: a, (off, flushed))

        total = off + flushed
        cntbuf[pl.ds(0, L)] = jnp.full((L,), 1, jnp.int32) * total
        pltpu.sync_copy(cntbuf, counts_hbm.at[pl.ds(_al8(c * L), L)])


@functools.partial(
    pl.kernel,
    out_type=(
        jax.ShapeDtypeStruct((N,), jnp.float32),
        jax.ShapeDtypeStruct((4 * N,), jnp.int32),
        jax.ShapeDtypeStruct((8,), jnp.int32),
    ),
    mesh=_MESH,
    compiler_params=_PARAMS,
    scratch_types=[
        pltpu.VMEM((NCH * L,), jnp.int32),
        pltpu.VMEM((NCH,), jnp.int32),
        pltpu.VMEM((PW,), jnp.int32),
        pltpu.VMEM((PW,), jnp.float32),
        pltpu.VMEM((PW,), jnp.int32),
        pltpu.VMEM((PW,), jnp.int32),
        pltpu.VMEM((PW,), jnp.int32),
        pltpu.VMEM((PW,), jnp.int32),
        pltpu.VMEM((PW,), jnp.int32),
        pltpu.VMEM((L,), jnp.int32),
    ],
)
def _place_kernel(cells_hbm, vals_hbm, counts_hbm,
                  feats_hbm, coords_hbm, total_hbm,
                  cbuf, offsv, sidx, fblk, cg, o4b, o4x, o4y, o4z, totbuf):
    w = _wid()
    lane = _lane()
    salt = w * 64

    pltpu.sync_copy(counts_hbm, cbuf)

    carry = jnp.int32(0)
    for k in range(NCH // L):
        cidx = (k * L + lane) * L
        cnt = plsc.load_gather(cbuf, [cidx])
        inc = plsc.cumsum(cnt)
        offsv[pl.ds(k * L, L)] = inc - cnt + carry
        carry = carry + inc[L - 1]
    tot = carry

    @pl.when(w == 0)
    def _():
        totbuf[pl.ds(0, L)] = jnp.full((L,), 1, jnp.int32) * tot
        pltpu.sync_copy(totbuf.at[pl.ds(0, 8)], total_hbm)

    base = jnp.minimum(w * PW, N - PW)

    def rbody(j, vcarry):
        p = base + j * L + lane
        lo = jnp.zeros((L,), jnp.int32)
        for step in (32, 16, 8, 4, 2, 1):
            cand = lo + step
            ov = plsc.load_gather(offsv, [jnp.minimum(cand, NCH - 1)])
            ok = (cand <= NCH - 1) & (ov <= p)
            lo = jnp.where(ok, cand, lo)
        obase = plsc.load_gather(offsv, [lo])
        s = lo * CH + (p - obase)
        s = jnp.where(p < tot, s, (p + salt) & 2047)
        sidx[pl.ds(j * L, L)] = s
        return vcarry

    lax.fori_loop(0, PW // L, rbody, jnp.int32(0))

    pltpu.sync_copy(cells_hbm.at[sidx], cg)
    pltpu.sync_copy(vals_hbm.at[sidx], fblk)

    def dbody(j, vcarry):
        p = base + j * L + lane
        live = p < tot
        cell = cg[pl.ds(j * L, L)]
        val = fblk[pl.ds(j * L, L)]
        fblk[pl.ds(j * L, L)] = jnp.where(live, val, 0.0)
        cell = jnp.where(live, cell, 0)
        o4b[pl.ds(j * L, L)] = cell >> 15
        o4x[pl.ds(j * L, L)] = (cell >> 10) & 31
        o4y[pl.ds(j * L, L)] = (cell >> 5) & 31
        o4z[pl.ds(j * L, L)] = cell & 31
        return vcarry

    lax.fori_loop(0, PW // L, dbody, jnp.int32(0))

    pltpu.sync_copy(fblk, feats_hbm.at[pl.ds(_al8(base), PW)])
    pltpu.sync_copy(o4b, coords_hbm.at[pl.ds(_al8(base), PW)])
    pltpu.sync_copy(o4x, coords_hbm.at[pl.ds(_al8(N + base), PW)])
    pltpu.sync_copy(o4y, coords_hbm.at[pl.ds(_al8(2 * N + base), PW)])
    pltpu.sync_copy(o4z, coords_hbm.at[pl.ds(_al8(3 * N + base), PW)])


def kernel(ghost_coords, ghost_feats, tensor_stride):
    del tensor_stride
    coords_flat = ghost_coords.astype(jnp.int32).reshape(4 * N)
    feats = ghost_feats.reshape(N).astype(jnp.float32)
    neg = jnp.full((CH,), -1.0, jnp.float32)

    cells, vals, counts, _ = _pool_kernel(coords_flat, feats, neg)
    feats_o, coords_cm, total = _place_kernel(cells, vals, counts)

    coords_o = coords_cm.reshape(4, N).T
    valid = jnp.arange(N, dtype=jnp.int32) < total[0]
    return feats_o.reshape(N, 1), coords_o, valid


# speedup vs baseline: 1.9012x; 1.3497x over previous
import functools

import jax
import jax.numpy as jnp
from jax import lax
from jax.experimental import pallas as pl
from jax.experimental.pallas import tpu as pltpu
from jax.experimental.pallas import tpu_sc as plsc

N = 100000
NC, NS, L = 2, 16, 16
W = NC * NS
PW = 3136
CELLS = 1 << 22
NCH = 64
CH = CELLS // NCH
BLK = 4000
NBLK = N // BLK
FLUSH = 8192
SEGV = 256
SEGC = SEGV * L
HS = 1568
HP = 4 * HS

_MESH = plsc.VectorSubcoreMesh(
    core_axis_name="c", subcore_axis_name="s", num_cores=NC, num_subcores=NS)
_PARAMS = pltpu.CompilerParams(needs_layout_passes=False)


def _al8(x):
    return pl.multiple_of(x, 8)


def _wid():
    return lax.axis_index("s") * NC + lax.axis_index("c")


def _lane():
    return lax.iota(jnp.int32, L)


@functools.partial(
    pl.kernel,
    out_type=(
        jax.ShapeDtypeStruct((CELLS,), jnp.int32),
        jax.ShapeDtypeStruct((CELLS,), jnp.float32),
        jax.ShapeDtypeStruct((NCH * L,), jnp.int32),
        jax.ShapeDtypeStruct((2 * N,), jnp.int32),
    ),
    mesh=_MESH,
    compiler_params=_PARAMS,
    scratch_types=[
        pltpu.VMEM((CH,), jnp.float32),
        pltpu.VMEM((HS,), jnp.int32),
        pltpu.VMEM((HS,), jnp.int32),
        pltpu.VMEM((HS,), jnp.int32),
        pltpu.VMEM((HS,), jnp.int32),
        pltpu.VMEM((HS,), jnp.int32),
        pltpu.VMEM((BLK,), jnp.int32),
        pltpu.VMEM((BLK,), jnp.float32),
        pltpu.VMEM((BLK,), jnp.int32),
        pltpu.VMEM((BLK,), jnp.float32),
        pltpu.VMEM((BLK + L,), jnp.int32),
        pltpu.VMEM((BLK + L,), jnp.float32),
        pltpu.VMEM((FLUSH + SEGC + L,), jnp.int32),
        pltpu.VMEM((FLUSH + SEGC + L,), jnp.float32),
        pltpu.VMEM((L,), jnp.int32),
        pltpu.SemaphoreType.DMA,
        pltpu.SemaphoreType.DMA,
    ],
)
def _pool_kernel(coords_hbm, f_hbm, neg_hbm, cells_hbm, vals_hbm, counts_hbm,
                 hc2_hbm, table, cbb, cxb, cyb, czb, hslab,
                 hbuf0, fbuf0, hbuf1, fbuf1,
                 fxc, fxv, ocell, oval, cntbuf, sem0, sem1):
    w = _wid()
    lane = _lane()
    sid = lax.axis_index("s")
    cid = lax.axis_index("c")
    hcbase = cid * N

    for s in range(4):
        rbase = jnp.minimum(sid * HP + s * HS, N - HS)
        pltpu.sync_copy(coords_hbm.at[pl.ds(_al8(rbase), HS)], cbb)
        pltpu.sync_copy(coords_hbm.at[pl.ds(_al8(N + rbase), HS)], cxb)
        pltpu.sync_copy(coords_hbm.at[pl.ds(_al8(2 * N + rbase), HS)], cyb)
        pltpu.sync_copy(coords_hbm.at[pl.ds(_al8(3 * N + rbase), HS)], czb)

        def hbody(j, hcarry):
            b = cbb[pl.ds(j * L, L)]
            x = cxb[pl.ds(j * L, L)]
            y = cyb[pl.ds(j * L, L)]
            z = czb[pl.ds(j * L, L)]
            hc = (b << 15) | ((x >> 2) << 10) | ((y >> 2) << 5) | (z >> 2)
            hslab[pl.ds(j * L, L)] = hc
            return hcarry

        lax.fori_loop(0, HS // L, hbody, jnp.int32(0))
        pltpu.sync_copy(hslab, hc2_hbm.at[pl.ds(_al8(hcbase + rbase), HS)])
    plsc.subcore_barrier()

    for t in range(2):
        c = w * 2 + t
        cbase = c * CH
        pltpu.sync_copy(neg_hbm, table)

        def _start(blk, hb, fb, sem):
            pltpu.async_copy(
                hc2_hbm.at[pl.ds(_al8(hcbase + blk * BLK), BLK)], hb, sem)
            pltpu.async_copy(f_hbm.at[pl.ds(_al8(blk * BLK), BLK)], fb, sem)

        def _wait(blk, hb, fb, sem):
            pltpu.make_async_copy(
                hc2_hbm.at[pl.ds(_al8(hcbase + blk * BLK), BLK)], hb,
                sem).wait()
            pltpu.make_async_copy(
                f_hbm.at[pl.ds(_al8(blk * BLK), BLK)], fb, sem).wait()

        def _process(hb, fb):
            @plsc.parallel_loop(0, BLK // L, step=1, unroll=4)
            def p12(j):
                h = hb[pl.ds(j * L, L)]
                v = fb[pl.ds(j * L, L)]
                inr = (h >> 16) == c
                li = jnp.where(inr, h & 0xFFFF, 0)
                cur = plsc.load_gather(table, [li], mask=inr)
                need = inr & (v > cur)
                plsc.store_scatter(table, [li], v, mask=need)

            @plsc.parallel_loop(0, BLK // L, step=1, unroll=2,
                                carry=jnp.int32(0))
            def p3(j, fcnt):
                h = hb[pl.ds(j * L, L)]
                v = fb[pl.ds(j * L, L)]
                inr = (h >> 16) == c
                li = jnp.where(inr, h & 0xFFFF, 0)
                cur = plsc.load_gather(table, [li], mask=inr)
                lost = inr & (v > cur)
                plsc.store_compressed(fxc.at[pl.ds(fcnt, L)], li, mask=lost)
                plsc.store_compressed(fxv.at[pl.ds(fcnt, L)], v, mask=lost)
                pc = plsc.all_reduce_population_count(lost)
                return fcnt + pc[0]

            fcnt = p3

            def drain(nfv):
                def dvbody(q, vcarry):
                    m = (q * L + lane) < fcnt
                    li = fxc[pl.ds(q * L, L)]
                    v = fxv[pl.ds(q * L, L)]
                    li = jnp.where(m, li, 0)

                    def wcond(pend):
                        return pend

                    def wbody(pend):
                        cur = plsc.load_gather(table, [li], mask=m)
                        need = m & (v > cur)
                        plsc.store_scatter(table, [li], v, mask=need)
                        cur2 = plsc.load_gather(table, [li], mask=m)
                        return jnp.any(m & (v > cur2))

                    lax.while_loop(wcond, wbody, jnp.any(m))
                    return vcarry

                lax.fori_loop(0, nfv, dvbody, jnp.int32(0))
                return jnp.int32(0)

            lax.cond(fcnt > 0, drain, lambda a: a, (fcnt + L - 1) // L)

        NPAIR = (NBLK - 1) // 2
        _start(0, hbuf0, fbuf0, sem0)

        def pairbody(p, carry):
            b0 = 2 * p
            _wait(b0, hbuf0, fbuf0, sem0)
            _start(b0 + 1, hbuf1, fbuf1, sem1)
            _process(hbuf0, fbuf0)
            _wait(b0 + 1, hbuf1, fbuf1, sem1)
            _start(b0 + 2, hbuf0, fbuf0, sem0)
            _process(hbuf1, fbuf1)
            return carry

        lax.fori_loop(0, NPAIR, pairbody, jnp.int32(0))
        _wait(NBLK - 1, hbuf0, fbuf0, sem0)
        _process(hbuf0, fbuf0)

        def segloop(s, carry):
            off0, flushed0 = carry

            @plsc.parallel_loop(0, SEGV, step=1, unroll=4, carry=off0)
            def seg(i, off_):
                idx = s * SEGV + i
                tv = table[pl.ds(idx * L, L)]
                occ = tv > -0.5
                cells = cbase + idx * L + lane
                plsc.store_compressed(ocell.at[pl.ds(off_, L)], cells,
                                      mask=occ)
                plsc.store_compressed(oval.at[pl.ds(off_, L)], tv, mask=occ)
                pc = plsc.all_reduce_population_count(occ)
                return off_ + pc[0]

            def do_flush(args):
                off_, flushed_ = args
                pltpu.sync_copy(ocell.at[pl.ds(0, FLUSH)],
                                cells_hbm.at[pl.ds(_al8(cbase + flushed_),
                                                   FLUSH)])
                pltpu.sync_copy(oval.at[pl.ds(0, FLUSH)],
                                vals_hbm.at[pl.ds(_al8(cbase + flushed_),
                                                  FLUSH)])
                rem = off_ - FLUSH

                def mv(q, mcarry):
                    tc = ocell[pl.ds(FLUSH + q * L, L)]
                    tvv = oval[pl.ds(FLUSH + q * L, L)]
                    ocell[pl.ds(q * L, L)] = tc
                    oval[pl.ds(q * L, L)] = tvv
                    return mcarry

                lax.fori_loop(0, (rem + L - 1) // L, mv, jnp.int32(0))
                return (rem, flushed_ + FLUSH)

            return lax.cond(seg >= FLUSH, do_flush, lambda a: a,
                            (seg, flushed0))

        off, flushed = lax.fori_loop(
            0, (CH // L) // SEGV, segloop, (jnp.int32(0), jnp.int32(0)))

        def final_flush(args):
            off_, flushed_ = args
            pltpu.sync_copy(ocell.at[pl.ds(0, FLUSH)],
                            cells_hbm.at[pl.ds(_al8(cbase + flushed_), FLUSH)])
            pltpu.sync_copy(oval.at[pl.ds(0, FLUSH)],
                            vals_hbm.at[pl.ds(_al8(cbase + flushed_), FLUSH)])
            return args

        lax.cond(off > 0, final_flush, lambda a: a, (off, flushed))

        total = off + flushed
        cntbuf[pl.ds(0, L)] = jnp.full((L,), 1, jnp.int32) * total
        pltpu.sync_copy(cntbuf, counts_hbm.at[pl.ds(_al8(c * L), L)])


@functools.partial(
    pl.kernel,
    out_type=(
        jax.ShapeDtypeStruct((N,), jnp.float32),
        jax.ShapeDtypeStruct((4 * N,), jnp.int32),
        jax.ShapeDtypeStruct((8,), jnp.int32),
    ),
    mesh=_MESH,
    compiler_params=_PARAMS,
    scratch_types=[
        pltpu.VMEM((NCH * L,), jnp.int32),
        pltpu.VMEM((NCH,), jnp.int32),
        pltpu.VMEM((PW,), jnp.int32),
        pltpu.VMEM((PW,), jnp.float32),
        pltpu.VMEM((PW,), jnp.int32),
        pltpu.VMEM((PW,), jnp.int32),
        pltpu.VMEM((PW,), jnp.int32),
        pltpu.VMEM((PW,), jnp.int32),
        pltpu.VMEM((PW,), jnp.int32),
        pltpu.VMEM((L,), jnp.int32),
    ],
)
def _place_kernel(cells_hbm, vals_hbm, counts_hbm,
                  feats_hbm, coords_hbm, total_hbm,
                  cbuf, offsv, sidx, fblk, cg, o4b, o4x, o4y, o4z, totbuf):
    w = _wid()
    lane = _lane()
    salt = w * 64

    pltpu.sync_copy(counts_hbm, cbuf)

    carry = jnp.int32(0)
    for k in range(NCH // L):
        cidx = (k * L + lane) * L
        cnt = plsc.load_gather(cbuf, [cidx])
        inc = plsc.cumsum(cnt)
        offsv[pl.ds(k * L, L)] = inc - cnt + carry
        carry = carry + inc[L - 1]
    tot = carry

    @pl.when(w == 0)
    def _():
        totbuf[pl.ds(0, L)] = jnp.full((L,), 1, jnp.int32) * tot
        pltpu.sync_copy(totbuf.at[pl.ds(0, 8)], total_hbm)

    base = jnp.minimum(w * PW, N - PW)

    def rbody(j, vcarry):
        p = base + j * L + lane
        lo = jnp.zeros((L,), jnp.int32)
        for step in (32, 16, 8, 4, 2, 1):
            cand = lo + step
            ov = plsc.load_gather(offsv, [jnp.minimum(cand, NCH - 1)])
            ok = (cand <= NCH - 1) & (ov <= p)
            lo = jnp.where(ok, cand, lo)
        obase = plsc.load_gather(offsv, [lo])
        s = lo * CH + (p - obase)
        s = jnp.where(p < tot, s, (p + salt) & 2047)
        sidx[pl.ds(j * L, L)] = s
        return vcarry

    lax.fori_loop(0, PW // L, rbody, jnp.int32(0))

    pltpu.sync_copy(cells_hbm.at[sidx], cg)
    pltpu.sync_copy(vals_hbm.at[sidx], fblk)

    def dbody(j, vcarry):
        p = base + j * L + lane
        live = p < tot
        cell = cg[pl.ds(j * L, L)]
        val = fblk[pl.ds(j * L, L)]
        fblk[pl.ds(j * L, L)] = jnp.where(live, val, 0.0)
        cell = jnp.where(live, cell, 0)
        o4b[pl.ds(j * L, L)] = cell >> 15
        o4x[pl.ds(j * L, L)] = (cell >> 10) & 31
        o4y[pl.ds(j * L, L)] = (cell >> 5) & 31
        o4z[pl.ds(j * L, L)] = cell & 31
        return vcarry

    lax.fori_loop(0, PW // L, dbody, jnp.int32(0))

    pltpu.sync_copy(fblk, feats_hbm.at[pl.ds(_al8(base), PW)])
    pltpu.sync_copy(o4b, coords_hbm.at[pl.ds(_al8(base), PW)])
    pltpu.sync_copy(o4x, coords_hbm.at[pl.ds(_al8(N + base), PW)])
    pltpu.sync_copy(o4y, coords_hbm.at[pl.ds(_al8(2 * N + base), PW)])
    pltpu.sync_copy(o4z, coords_hbm.at[pl.ds(_al8(3 * N + base), PW)])


def kernel(ghost_coords, ghost_feats, tensor_stride):
    del tensor_stride
    coords_cm = ghost_coords.astype(jnp.int32).T.reshape(4 * N)
    feats = ghost_feats.reshape(N).astype(jnp.float32)
    neg = jnp.full((CH,), -1.0, jnp.float32)

    cells, vals, counts, _ = _pool_kernel(coords_cm, feats, neg)
    feats_o, coords_cm, total = _place_kernel(cells, vals, counts)

    coords_o = coords_cm.reshape(4, N).T
    valid = jnp.arange(N, dtype=jnp.int32) < total[0]
    return feats_o.reshape(N, 1), coords_o, valid
